# baseline (device time: 136579 ns/iter reference)
import functools

import jax
import jax.numpy as jnp
from jax import lax
from jax.experimental import pallas as pl
from jax.experimental.pallas import tpu as pltpu

N_DEV = 16
N_STAGES = 4


def kernel(x, router_W, route_idx, expert_W, shared_W):
    n_tok, d = x.shape
    n_exp_total = router_W.shape[1]
    e_loc = expert_W.shape[0]
    h = shared_W.shape[1]

    def body(x_ref, rw_ref, idx_ref, ew_ref, sw_ref, out_ref,
             acc_ref, recv_ref, send_sems, recv_sems):
        me = lax.axis_index("i")

        barrier_sem = pltpu.get_barrier_semaphore()
        for k in range(N_STAGES):
            partner = me ^ (1 << k)
            pl.semaphore_signal(
                barrier_sem, inc=1,
                device_id=(partner,), device_id_type=pl.DeviceIdType.MESH,
            )
        pl.semaphore_wait(barrier_sem, N_STAGES)

        xv = x_ref[...]
        scores = jnp.dot(xv, rw_ref[...], preferred_element_type=jnp.float32)
        m = jnp.max(scores, axis=-1, keepdims=True)
        e = jnp.exp(scores - m)
        probs = e / jnp.sum(e, axis=-1, keepdims=True)
        idx = idx_ref[...]
        onehot = (idx == lax.broadcasted_iota(jnp.int32, (n_tok, n_exp_total), 1))
        p_tok = jnp.sum(probs * onehot.astype(jnp.float32), axis=-1)

        acc = jnp.zeros((n_tok, h), jnp.float32)
        for el in range(e_loc):
            ge = me * e_loc + el
            coef = jnp.where(idx[:, 0] == ge, p_tok, 0.0)
            acc = acc + jnp.dot(
                xv * coef[:, None], ew_ref[el],
                preferred_element_type=jnp.float32,
            )
        acc_ref[...] = acc

        for k in range(N_STAGES):
            partner = me ^ (1 << k)
            rdma = pltpu.make_async_remote_copy(
                src_ref=acc_ref,
                dst_ref=recv_ref.at[k],
                send_sem=send_sems.at[k],
                recv_sem=recv_sems.at[k],
                device_id=(partner,),
                device_id_type=pl.DeviceIdType.MESH,
            )
            rdma.start()
            rdma.wait()
            acc_ref[...] = acc_ref[...] + recv_ref[k]

        out_ref[...] = acc_ref[...] + jnp.dot(
            xv, sw_ref[...], preferred_element_type=jnp.float32
        )

        @functools.partial(
            pl.run_scoped, second_barrier=pltpu.SemaphoreType.REGULAR
        )
        def _(second_barrier):
            for k in range(N_STAGES):
                partner = me ^ (1 << k)
                pl.semaphore_signal(
                    second_barrier, inc=1,
                    device_id=(partner,), device_id_type=pl.DeviceIdType.MESH,
                )
            pl.semaphore_wait(second_barrier, N_STAGES)

    return pl.pallas_call(
        body,
        out_shape=jax.ShapeDtypeStruct((n_tok, h), jnp.float32),
        in_specs=[pl.BlockSpec(memory_space=pltpu.VMEM)] * 5,
        out_specs=pl.BlockSpec(memory_space=pltpu.VMEM),
        scratch_shapes=[
            pltpu.VMEM((n_tok, h), jnp.float32),
            pltpu.VMEM((N_STAGES, n_tok, h), jnp.float32),
            pltpu.SemaphoreType.DMA((N_STAGES,)),
            pltpu.SemaphoreType.DMA((N_STAGES,)),
        ],
        compiler_params=pltpu.CompilerParams(collective_id=0),
    )(x, router_W, route_idx, expert_W, shared_W)


# device time: 74725 ns/iter; 1.8278x vs baseline; 1.8278x over previous
import functools

import jax
import jax.numpy as jnp
from jax import lax
from jax.experimental import pallas as pl
from jax.experimental.pallas import tpu as pltpu

N_DEV = 16
N_STAGES = 4


def kernel(x, router_W, route_idx, expert_W, shared_W):
    n_tok, d = x.shape
    n_exp_total = router_W.shape[1]
    e_loc = expert_W.shape[0]
    h = shared_W.shape[1]

    def body(x_ref, rw_ref, idx_ref, ew_ref, sw_ref, out_ref,
             acc_ref, recv_ref, send_sems, recv_sems):
        me = lax.axis_index("i")

        barrier_sem = pltpu.get_barrier_semaphore()
        for k in range(N_STAGES):
            partner = me ^ (1 << k)
            pl.semaphore_signal(
                barrier_sem, inc=1,
                device_id=(partner,), device_id_type=pl.DeviceIdType.MESH,
            )
        pl.semaphore_wait(barrier_sem, N_STAGES)

        xv = x_ref[...]
        scores = jnp.dot(xv, rw_ref[...], preferred_element_type=jnp.float32)
        m = jnp.max(scores, axis=-1, keepdims=True)
        e = jnp.exp(scores - m)
        probs = e / jnp.sum(e, axis=-1, keepdims=True)
        idx = idx_ref[...]
        onehot = (idx == lax.broadcasted_iota(jnp.int32, (n_tok, n_exp_total), 1))
        p_tok = jnp.sum(probs * onehot.astype(jnp.float32), axis=-1)

        acc = jnp.zeros((n_tok, h), jnp.float32)
        for el in range(e_loc):
            ge = me * e_loc + el
            coef = jnp.where(idx[:, 0] == ge, p_tok, 0.0)
            acc = acc + jnp.dot(
                xv * coef[:, None], ew_ref[el],
                preferred_element_type=jnp.float32,
            )
        acc_ref[...] = acc

        off = me * 0
        for k in range(N_STAGES):
            L = n_tok >> (k + 1)
            r0 = n_tok - (n_tok >> k)
            partner = me ^ (1 << k)
            bit = (me >> k) & 1
            send_off = off + (1 - bit) * L
            keep_off = off + bit * L
            rdma = pltpu.make_async_remote_copy(
                src_ref=acc_ref.at[pl.ds(send_off, L)],
                dst_ref=recv_ref.at[pl.ds(r0, L)],
                send_sem=send_sems.at[k],
                recv_sem=recv_sems.at[k],
                device_id=(partner,),
                device_id_type=pl.DeviceIdType.MESH,
            )
            rdma.start()
            rdma.wait()
            acc_ref[pl.ds(keep_off, L), :] = (
                acc_ref[pl.ds(keep_off, L), :] + recv_ref[pl.ds(r0, L), :]
            )
            off = keep_off

        seg = n_tok // N_DEV
        out_ref[pl.ds(off, seg), :] = acc_ref[pl.ds(off, seg), :] + jnp.dot(
            x_ref[pl.ds(off, seg), :], sw_ref[...],
            preferred_element_type=jnp.float32,
        )

        for j, k in enumerate(reversed(range(N_STAGES))):
            S = n_tok >> (k + 1)
            partner = me ^ (1 << k)
            bit = (me >> k) & 1
            rdma = pltpu.make_async_remote_copy(
                src_ref=out_ref.at[pl.ds(off, S)],
                dst_ref=out_ref.at[pl.ds(off, S)],
                send_sem=send_sems.at[N_STAGES + j],
                recv_sem=recv_sems.at[N_STAGES + j],
                device_id=(partner,),
                device_id_type=pl.DeviceIdType.MESH,
            )
            rdma.start()
            rdma.wait()
            off = off - bit * S

        @functools.partial(
            pl.run_scoped, second_barrier=pltpu.SemaphoreType.REGULAR
        )
        def _(second_barrier):
            for k in range(N_STAGES):
                partner = me ^ (1 << k)
                pl.semaphore_signal(
                    second_barrier, inc=1,
                    device_id=(partner,), device_id_type=pl.DeviceIdType.MESH,
                )
            pl.semaphore_wait(second_barrier, N_STAGES)

    return pl.pallas_call(
        body,
        out_shape=jax.ShapeDtypeStruct((n_tok, h), jnp.float32),
        in_specs=[pl.BlockSpec(memory_space=pltpu.VMEM)] * 5,
        out_specs=pl.BlockSpec(memory_space=pltpu.VMEM),
        scratch_shapes=[
            pltpu.VMEM((n_tok, h), jnp.float32),
            pltpu.VMEM((n_tok, h), jnp.float32),
            pltpu.SemaphoreType.DMA((2 * N_STAGES,)),
            pltpu.SemaphoreType.DMA((2 * N_STAGES,)),
        ],
        compiler_params=pltpu.CompilerParams(collective_id=0),
    )(x, router_W, route_idx, expert_W, shared_W)


# device time: 74427 ns/iter; 1.8351x vs baseline; 1.0040x over previous
import functools

import jax
import jax.numpy as jnp
from jax import lax
from jax.experimental import pallas as pl
from jax.experimental.pallas import tpu as pltpu

N_DEV = 16
N_STAGES = 4


def kernel(x, router_W, route_idx, expert_W, shared_W):
    n_tok, d = x.shape
    n_exp_total = router_W.shape[1]
    e_loc = expert_W.shape[0]
    h = shared_W.shape[1]

    def body(x_ref, rw_ref, idx_ref, ew_ref, sw_ref, out_ref,
             acc_ref, recv_ref, p_ref, send_sems, recv_sems):
        me = lax.axis_index("i")

        barrier_sem = pltpu.get_barrier_semaphore()
        for k in range(N_STAGES):
            partner = me ^ (1 << k)
            pl.semaphore_signal(
                barrier_sem, inc=1,
                device_id=(partner,), device_id_type=pl.DeviceIdType.MESH,
            )
        pl.semaphore_wait(barrier_sem, N_STAGES)

        xv = x_ref[...]
        scores = jnp.dot(xv, rw_ref[...], preferred_element_type=jnp.float32)
        m = jnp.max(scores, axis=-1, keepdims=True)
        e = jnp.exp(scores - m)
        probs = e / jnp.sum(e, axis=-1, keepdims=True)
        idx = idx_ref[...]
        onehot = (idx == lax.broadcasted_iota(jnp.int32, (n_tok, n_exp_total), 1))
        p_ref[...] = jnp.sum(probs * onehot.astype(jnp.float32), axis=-1,
                             keepdims=True)

        def masked_acc(row_off, nrows):
            xs = x_ref[pl.ds(row_off, nrows), :]
            idx_s = idx_ref[pl.ds(row_off, nrows), :]
            p_s = p_ref[pl.ds(row_off, nrows), :]
            a = jnp.zeros((nrows, h), jnp.float32)
            for el in range(e_loc):
                ge = me * e_loc + el
                coef = jnp.where(idx_s == ge, p_s, 0.0)
                a = a + jnp.dot(
                    xs * coef, ew_ref[el],
                    preferred_element_type=jnp.float32,
                )
            return a

        half = n_tok // 2
        bit0 = me & 1
        send0 = (1 - bit0) * half
        keep0 = bit0 * half
        acc_ref[pl.ds(send0, half), :] = masked_acc(send0, half)
        rdma0 = pltpu.make_async_remote_copy(
            src_ref=acc_ref.at[pl.ds(send0, half)],
            dst_ref=recv_ref.at[pl.ds(0, half)],
            send_sem=send_sems.at[0],
            recv_sem=recv_sems.at[0],
            device_id=(me ^ 1,),
            device_id_type=pl.DeviceIdType.MESH,
        )
        rdma0.start()
        acc_ref[pl.ds(keep0, half), :] = masked_acc(keep0, half)
        rdma0.wait()

        off = keep0
        for k in range(1, N_STAGES):
            L_prev = n_tok >> k
            L = n_tok >> (k + 1)
            r_prev = n_tok - (n_tok >> (k - 1))
            r_k = n_tok - (n_tok >> k)
            partner = me ^ (1 << k)
            bit = (me >> k) & 1
            send_off = off + (1 - bit) * L
            keep_off = off + bit * L
            acc_ref[pl.ds(send_off, L), :] = (
                acc_ref[pl.ds(send_off, L), :]
                + recv_ref[pl.ds(r_prev + (1 - bit) * L, L), :]
            )
            rdma = pltpu.make_async_remote_copy(
                src_ref=acc_ref.at[pl.ds(send_off, L)],
                dst_ref=recv_ref.at[pl.ds(r_k, L)],
                send_sem=send_sems.at[k],
                recv_sem=recv_sems.at[k],
                device_id=(partner,),
                device_id_type=pl.DeviceIdType.MESH,
            )
            rdma.start()
            acc_ref[pl.ds(keep_off, L), :] = (
                acc_ref[pl.ds(keep_off, L), :]
                + recv_ref[pl.ds(r_prev + bit * L, L), :]
            )
            rdma.wait()
            off = keep_off

        seg = n_tok // N_DEV
        r_last = n_tok - (n_tok >> (N_STAGES - 1))
        acc_ref[pl.ds(off, seg), :] = (
            acc_ref[pl.ds(off, seg), :] + recv_ref[pl.ds(r_last, seg), :]
        )

        seg = n_tok // N_DEV
        out_ref[pl.ds(off, seg), :] = acc_ref[pl.ds(off, seg), :] + jnp.dot(
            x_ref[pl.ds(off, seg), :], sw_ref[...],
            preferred_element_type=jnp.float32,
        )

        for j, k in enumerate(reversed(range(N_STAGES))):
            S = n_tok >> (k + 1)
            partner = me ^ (1 << k)
            bit = (me >> k) & 1
            rdma = pltpu.make_async_remote_copy(
                src_ref=out_ref.at[pl.ds(off, S)],
                dst_ref=out_ref.at[pl.ds(off, S)],
                send_sem=send_sems.at[N_STAGES + j],
                recv_sem=recv_sems.at[N_STAGES + j],
                device_id=(partner,),
                device_id_type=pl.DeviceIdType.MESH,
            )
            rdma.start()
            rdma.wait()
            off = off - bit * S

        @functools.partial(
            pl.run_scoped, second_barrier=pltpu.SemaphoreType.REGULAR
        )
        def _(second_barrier):
            for k in range(N_STAGES):
                partner = me ^ (1 << k)
                pl.semaphore_signal(
                    second_barrier, inc=1,
                    device_id=(partner,), device_id_type=pl.DeviceIdType.MESH,
                )
            pl.semaphore_wait(second_barrier, N_STAGES)

    return pl.pallas_call(
        body,
        out_shape=jax.ShapeDtypeStruct((n_tok, h), jnp.float32),
        in_specs=[pl.BlockSpec(memory_space=pltpu.VMEM)] * 5,
        out_specs=pl.BlockSpec(memory_space=pltpu.VMEM),
        scratch_shapes=[
            pltpu.VMEM((n_tok, h), jnp.float32),
            pltpu.VMEM((n_tok, h), jnp.float32),
            pltpu.VMEM((n_tok, 1), jnp.float32),
            pltpu.SemaphoreType.DMA((2 * N_STAGES,)),
            pltpu.SemaphoreType.DMA((2 * N_STAGES,)),
        ],
        compiler_params=pltpu.CompilerParams(collective_id=0),
    )(x, router_W, route_idx, expert_W, shared_W)


# device time: 54248 ns/iter; 2.5177x vs baseline; 1.3720x over previous
import functools

import jax
import jax.numpy as jnp
from jax import lax
from jax.experimental import pallas as pl
from jax.experimental.pallas import tpu as pltpu

N_DEV = 16
N_ROUNDS = 3
RS_SCHED = (("p4", 2, 3), (2, 3, "p4"))


def kernel(x, router_W, route_idx, expert_W, shared_W):
    n_tok, d = x.shape
    n_exp_total = router_W.shape[1]
    e_loc = expert_W.shape[0]
    h = shared_W.shape[1]
    hc = h // 2

    def body(x_ref, rw_ref, idx_ref, ew_ref, sw_ref, out_ref,
             acc_ref, recv_ref, gat_ref, p_ref, send_sems, recv_sems):
        me = lax.axis_index("i")
        q = me & 3
        plane0 = me - q

        def comm_partners():
            return [plane0 + ((q + dq) & 3) for dq in (1, 2, 3)] + [
                me ^ 4, me ^ 8]

        barrier_sem = pltpu.get_barrier_semaphore()
        for partner in comm_partners():
            pl.semaphore_signal(
                barrier_sem, inc=1,
                device_id=(partner,), device_id_type=pl.DeviceIdType.MESH,
            )
        pl.semaphore_wait(barrier_sem, 5)

        xv = x_ref[...]
        scores = jnp.dot(xv, rw_ref[...], preferred_element_type=jnp.float32)
        m = jnp.max(scores, axis=-1, keepdims=True)
        e = jnp.exp(scores - m)
        probs = e / jnp.sum(e, axis=-1, keepdims=True)
        idx = idx_ref[...]
        onehot = (idx == lax.broadcasted_iota(jnp.int32, (n_tok, n_exp_total), 1))
        p_ref[...] = jnp.sum(probs * onehot.astype(jnp.float32), axis=-1,
                             keepdims=True)

        def masked_acc_cols(c):
            a = jnp.zeros((n_tok, hc), jnp.float32)
            for el in range(e_loc):
                ge = me * e_loc + el
                coef = jnp.where(idx == ge, p_ref[...], 0.0)
                a = a + jnp.dot(
                    xv * coef, ew_ref[el, :, c * hc:(c + 1) * hc],
                    preferred_element_type=jnp.float32,
                )
            return a

        offs = [me * 0, me * 0]
        lens = [n_tok, n_tok]
        rbase = [0, 0]
        pending = [None, None]
        rd = [[], []]

        def start_rs(c, r):
            kind = RS_SCHED[c][r]
            if kind == "p4":
                L = lens[c] // 4
                rb = rbase[c]
                keep_off = offs[c] + q * L
                rdmas = []
                for i, dq in enumerate((1, 2, 3)):
                    qp = (q + dq) & 3
                    tgt = plane0 + qp
                    slot = (q - qp - 1) & 3
                    rdma = pltpu.make_async_remote_copy(
                        src_ref=acc_ref.at[c, pl.ds(offs[c] + qp * L, L)],
                        dst_ref=recv_ref.at[c, pl.ds(rb + slot * L, L)],
                        send_sem=send_sems.at[0, c, r, i],
                        recv_sem=recv_sems.at[0, c, r, i],
                        device_id=(tgt,),
                        device_id_type=pl.DeviceIdType.MESH,
                    )
                    rdma.start()
                    rdmas.append(rdma)
                rd[c] = rdmas
                pending[c] = ("p4", keep_off, L, rb)
                rbase[c] = rb + 3 * L
            else:
                k = kind
                L = lens[c] // 2
                rb = rbase[c]
                bit = (me >> k) & 1
                send_off = offs[c] + (1 - bit) * L
                keep_off = offs[c] + bit * L
                rdma = pltpu.make_async_remote_copy(
                    src_ref=acc_ref.at[c, pl.ds(send_off, L)],
                    dst_ref=recv_ref.at[c, pl.ds(rb, L)],
                    send_sem=send_sems.at[0, c, r, 0],
                    recv_sem=recv_sems.at[0, c, r, 0],
                    device_id=(me ^ (1 << k),),
                    device_id_type=pl.DeviceIdType.MESH,
                )
                rdma.start()
                rd[c] = [rdma]
                pending[c] = ("z", keep_off, L, rb)
                rbase[c] = rb + L
            offs[c] = keep_off
            lens[c] = L

        def finish_rs(c):
            for rdma in rd[c]:
                rdma.wait()
            kind, keep_off, L, rb = pending[c]
            kept = acc_ref[c, pl.ds(keep_off, L), :]
            if kind == "p4":
                kept = (kept
                        + recv_ref[c, pl.ds(rb, L), :]
                        + recv_ref[c, pl.ds(rb + L, L), :]
                        + recv_ref[c, pl.ds(rb + 2 * L, L), :])
            else:
                kept = kept + recv_ref[c, pl.ds(rb, L), :]
            acc_ref[c, pl.ds(keep_off, L), :] = kept

        def start_ag(c, r):
            kind = RS_SCHED[c][N_ROUNDS - 1 - r]
            S = lens[c]
            src = gat_ref.at[c, pl.ds(offs[c], S)]
            if kind == "p4":
                rdmas = []
                for i, dq in enumerate((1, 2, 3)):
                    qp = (q + dq) & 3
                    rdma = pltpu.make_async_remote_copy(
                        src_ref=src,
                        dst_ref=src,
                        send_sem=send_sems.at[1, c, r, i],
                        recv_sem=recv_sems.at[1, c, r, i],
                        device_id=(plane0 + qp,),
                        device_id_type=pl.DeviceIdType.MESH,
                    )
                    rdma.start()
                    rdmas.append(rdma)
                rd[c] = rdmas
                offs[c] = offs[c] - q * S
                lens[c] = 4 * S
            else:
                k = kind
                bit = (me >> k) & 1
                rdma = pltpu.make_async_remote_copy(
                    src_ref=src,
                    dst_ref=src,
                    send_sem=send_sems.at[1, c, r, 0],
                    recv_sem=recv_sems.at[1, c, r, 0],
                    device_id=(me ^ (1 << k),),
                    device_id_type=pl.DeviceIdType.MESH,
                )
                rdma.start()
                rd[c] = [rdma]
                offs[c] = offs[c] - bit * S
                lens[c] = 2 * S

        def finish_ag(c):
            for rdma in rd[c]:
                rdma.wait()

        acc_ref[0] = masked_acc_cols(0)
        start_rs(0, 0)
        acc_ref[1] = masked_acc_cols(1)
        start_rs(1, 0)

        for r in range(1, N_ROUNDS):
            for c in (0, 1):
                finish_rs(c)
                start_rs(c, r)

        seg = n_tok // N_DEV
        for c in (0, 1):
            finish_rs(c)
            gat_ref[c, pl.ds(offs[c], seg), :] = (
                acc_ref[c, pl.ds(offs[c], seg), :]
                + jnp.dot(
                    x_ref[pl.ds(offs[c], seg), :],
                    sw_ref[:, c * hc:(c + 1) * hc],
                    preferred_element_type=jnp.float32,
                )
            )
            start_ag(c, 0)

        for r in range(1, N_ROUNDS):
            for c in (0, 1):
                finish_ag(c)
                start_ag(c, r)
        for c in (0, 1):
            finish_ag(c)

        out_ref[:, 0:hc] = gat_ref[0]
        out_ref[:, hc:h] = gat_ref[1]

        @functools.partial(
            pl.run_scoped, second_barrier=pltpu.SemaphoreType.REGULAR
        )
        def _(second_barrier):
            for partner in comm_partners():
                pl.semaphore_signal(
                    second_barrier, inc=1,
                    device_id=(partner,), device_id_type=pl.DeviceIdType.MESH,
                )
            pl.semaphore_wait(second_barrier, 5)

    return pl.pallas_call(
        body,
        out_shape=jax.ShapeDtypeStruct((n_tok, h), jnp.float32),
        in_specs=[pl.BlockSpec(memory_space=pltpu.VMEM)] * 5,
        out_specs=pl.BlockSpec(memory_space=pltpu.VMEM),
        scratch_shapes=[
            pltpu.VMEM((2, n_tok, hc), jnp.float32),
            pltpu.VMEM((2, n_tok, hc), jnp.float32),
            pltpu.VMEM((2, n_tok, hc), jnp.float32),
            pltpu.VMEM((n_tok, 1), jnp.float32),
            pltpu.SemaphoreType.DMA((2, 2, N_ROUNDS, 3)),
            pltpu.SemaphoreType.DMA((2, 2, N_ROUNDS, 3)),
        ],
        compiler_params=pltpu.CompilerParams(collective_id=0),
    )(x, router_W, route_idx, expert_W, shared_W)
